# baseline (speedup 1.0000x reference)
import functools

import numpy as np
import jax
import jax.numpy as jnp
from jax import lax
from jax.experimental import pallas as pl
from jax.experimental.pallas import tpu as pltpu
from jax.experimental.pallas import tpu_sc as plsc

H = 8
L = 4
P = 4
LV = 21760
N = 4
LQ = 900
LQP = 960
C = 256
CH = 32
NW = 32
PAIRS_PER_W = (N * LQP) // NW
CHUNK = 8
NCHUNK = PAIRS_PER_W // CHUNK
GRAN = 1
GPC = CHUNK // GRAN
ROWS = H * L * P * 4


def _prep_body(qT, Wa, Wox, Woy, ba, box, boy, cx, cy, ww, hb, iref, wref):
    n = pl.program_id(0)
    j = pl.program_id(1)
    q = qT[0]
    lrow = lax.broadcasted_iota(jnp.int32, (L * P, LQP), 0) // P
    Wl = jnp.where(lrow == 0, 128, jnp.where(lrow == 1, 64,
                                             jnp.where(lrow == 2, 32, 16)))
    offl = jnp.where(lrow == 0, 0, jnp.where(lrow == 1, 16384,
                                             jnp.where(lrow == 2, 20480, 21504)))
    Wlf = Wl.astype(jnp.float32)
    for hh in range(2):
        att = (jnp.dot(Wa[hh], q, preferred_element_type=jnp.float32)
               + ba[hh][:, 0:1])
        att = att - jnp.max(att, axis=0, keepdims=True)
        att = jnp.exp(att)
        att = att / jnp.sum(att, axis=0, keepdims=True)
        ox = (jnp.dot(Wox[hh], q, preferred_element_type=jnp.float32)
              + box[hh][:, 0:1])
        oy = (jnp.dot(Woy[hh], q, preferred_element_type=jnp.float32)
              + boy[hh][:, 0:1])
        locx = cx[0] + ox * 0.125 * ww[0]
        locy = cy[0] + oy * 0.125 * hb[0]
        x = locx * Wlf - 0.5
        y = locy * Wlf - 0.5
        x0 = jnp.floor(x)
        y0 = jnp.floor(y)
        fx = x - x0
        fy = y - y0
        base = n * (LV * H) + 2 * j + hh
        ci = 0
        for dy in (0, 1):
            wy = fy if dy else 1.0 - fy
            for dx in (0, 1):
                wx = fx if dx else 1.0 - fx
                xi = x0 + dx
                yi = y0 + dy
                valid = ((xi >= 0) & (xi <= Wlf - 1.0)
                         & (yi >= 0) & (yi <= Wlf - 1.0))
                xc = jnp.clip(xi, 0.0, Wlf - 1.0).astype(jnp.int32)
                yc = jnp.clip(yi, 0.0, Wlf - 1.0).astype(jnp.int32)
                rowid = (offl + yc * Wl + xc) * H + base
                wcorn = att * (wx * wy) * valid.astype(jnp.float32)
                lo = hh * 64 + ci * (L * P)
                iref[0, :, lo:lo + L * P] = jnp.transpose(rowid)
                wref[0, :, lo:lo + L * P] = jnp.transpose(wcorn)
                ci += 1


def _mm_body(x, w, b, o):
    o[...] = (jnp.dot(x[...], w[...], preferred_element_type=jnp.float32)
              + b[...]).astype(o.dtype)


def _matmul(x, w, b, bm, out_dtype=jnp.float32):
    m = x.shape[0]
    assert m % bm == 0
    return pl.pallas_call(
        _mm_body,
        grid=(m // bm,),
        in_specs=[
            pl.BlockSpec((bm, x.shape[1]), lambda i: (i, 0)),
            pl.BlockSpec((w.shape[0], w.shape[1]), lambda i: (0, 0)),
            pl.BlockSpec((1, w.shape[1]), lambda i: (0, 0)),
        ],
        out_specs=pl.BlockSpec((bm, w.shape[1]), lambda i: (i, 0)),
        out_shape=jax.ShapeDtypeStruct((m, w.shape[1]), out_dtype),
    )(x, w, b.reshape(1, -1))


def _sc_body(vtab, idxh, wgth, outh, ibuf, wbuf, rows, ob,
             i0, i1, g0, g1, o0, o1):
    isems = (i0, i1)
    gsems = (g0, g1)
    osems = (o0, o1)
    wid = lax.axis_index("s") * 2 + lax.axis_index("c")
    base = wid * PAIRS_PER_W
    zero = jnp.zeros((16,), jnp.float32)

    def load_chunk(c, cb):
        pltpu.async_copy(idxh.at[pl.ds(base + c * CHUNK, CHUNK)],
                         ibuf.at[cb], isems[cb])
        pltpu.async_copy(wgth.at[pl.ds(base + c * CHUNK, CHUNK)],
                         wbuf.at[cb], isems[cb])

    def drain_chunk(cb):
        pltpu.make_async_copy(idxh.at[pl.ds(0, CHUNK)],
                              ibuf.at[cb], isems[cb]).wait()
        pltpu.make_async_copy(wgth.at[pl.ds(0, CHUNK)],
                              wbuf.at[cb], isems[cb]).wait()

    def fire(g, cb, rb):
        for u in range(GRAN):
            tin = g * GRAN + u
            for jj in range(4):
                pltpu.async_copy(
                    vtab.at[ibuf.at[cb, tin, jj]],
                    rows.at[rb].at[pl.ds((u * 4 + jj) * 128, 128)],
                    gsems[rb])

    def drain_gather(rb):
        pltpu.make_async_copy(vtab.at[pl.ds(0, GRAN * ROWS)],
                              rows.at[rb], gsems[rb]).wait()

    def drain_out(b):
        pltpu.make_async_copy(outh.at[pl.ds(0, GRAN)], ob.at[b],
                              osems[b]).wait()

    def compute(g, c, cb, rb):
        for u in range(GRAN):
            tin = g * GRAN + u
            for i in range(C // 16):
                ob[rb, u, pl.ds(i * 16, 16)] = zero

            def grp_body(gg, carry, u=u, tin=tin):
                wvec = wbuf[cb, tin, pl.ds(gg * 16, 16)]
                a0 = zero
                a1 = zero
                for k in range(16):
                    r = u * ROWS + gg * 16 + k
                    e, o = plsc.unpack(rows[rb, r, 0:32],
                                       format=plsc.PackFormat.INTERLEAVED,
                                       preferred_element_type=jnp.float32)
                    a0 = a0 + wvec[k] * e
                    a1 = a1 + wvec[k] * o
                h = gg // 4
                ob[rb, u, pl.ds(h * CH, 16)] = (ob[rb, u, pl.ds(h * CH, 16)]
                                                + a0)
                ob[rb, u, pl.ds(h * CH + 16, 16)] = (
                    ob[rb, u, pl.ds(h * CH + 16, 16)] + a1)
                return carry

            lax.fori_loop(0, ROWS // 16, grp_body, 0)
            pltpu.async_copy(ob.at[rb, u],
                             outh.at[base + c * CHUNK + tin], osems[rb])

    def chunk_pairs(c, cb, last):
        for g in range(GPC):
            rb = g % 2
            if g < GPC - 1:
                fire(g + 1, cb, (g + 1) % 2)
            drain_gather(rb)
            drain_out(rb)
            compute(g, c, cb, rb)
        if not last:
            drain_chunk(cb ^ 1)
            fire(0, cb ^ 1, 0)

    for b in range(2):
        pltpu.async_copy(outh.at[pl.ds(0, GRAN)], ob.at[b], osems[b])
    load_chunk(0, 0)
    drain_chunk(0)
    fire(0, 0, 0)

    def dchunk_body(g, carry):
        load_chunk(2 * g + 1, 1)
        chunk_pairs(2 * g, 0, False)
        load_chunk(2 * g + 2, 0)
        chunk_pairs(2 * g + 1, 1, False)
        return carry

    lax.fori_loop(0, (NCHUNK - 1) // 2, dchunk_body, 0)
    chunk_pairs(NCHUNK - 1, 0, True)
    drain_out(0)
    drain_out(1)


def _sc_gather(vtab, idx_t, wgt_t):
    k = functools.partial(
        pl.kernel,
        out_type=jax.ShapeDtypeStruct((N * LQP, C), jnp.float32),
        mesh=plsc.VectorSubcoreMesh(core_axis_name="c", subcore_axis_name="s"),
        scratch_types=[
            pltpu.VMEM((2, CHUNK, 4, 128), jnp.int32),
            pltpu.VMEM((2, CHUNK, ROWS), jnp.float32),
            pltpu.VMEM((2, GRAN * ROWS, CH), jnp.bfloat16),
            pltpu.VMEM((2, GRAN, C), jnp.float32),
            pltpu.SemaphoreType.DMA,
            pltpu.SemaphoreType.DMA,
            pltpu.SemaphoreType.DMA,
            pltpu.SemaphoreType.DMA,
            pltpu.SemaphoreType.DMA,
            pltpu.SemaphoreType.DMA,
        ],
        compiler_params=pltpu.CompilerParams(use_tc_tiling_on_sc=False,
                                            needs_layout_passes=False),
    )(_sc_body)
    return k(vtab, idx_t, wgt_t)


def kernel(query, ref_points, value, value_spatial_shapes,
           W_off, b_off, W_attn, b_attn, W_val, b_val, W_out, b_out):
    qT = jnp.pad(query, ((0, 0), (0, LQP - LQ), (0, 0))).transpose(0, 2, 1)
    WaT = W_attn.T.reshape(H, L * P, C)
    WoT = W_off.T.reshape(H, L * P, 2, C)
    WoxT = WoT[:, :, 0, :]
    WoyT = WoT[:, :, 1, :]
    baT = jnp.broadcast_to(b_attn.reshape(H, L * P, 1), (H, L * P, 128))
    bo = b_off.reshape(H, L * P, 2)
    boxT = jnp.broadcast_to(bo[:, :, 0:1], (H, L * P, 128))
    boyT = jnp.broadcast_to(bo[:, :, 1:2], (H, L * P, 128))
    rp = jnp.pad(ref_points,
                 ((0, 0), (0, LQP - LQ), (0, 0), (0, 0))).transpose(0, 2, 3, 1)
    cxb = jnp.repeat(rp[:, :, 0, :], P, axis=1)
    cyb = jnp.repeat(rp[:, :, 1, :], P, axis=1)
    wwb = jnp.repeat(rp[:, :, 2, :], P, axis=1)
    hhb = jnp.repeat(rp[:, :, 3, :], P, axis=1)

    spec_q = pl.BlockSpec((1, C, LQP), lambda n, j: (n, 0, 0))
    spec_w = pl.BlockSpec((2, L * P, C), lambda n, j: (j, 0, 0))
    spec_b = pl.BlockSpec((2, L * P, 128), lambda n, j: (j, 0, 0))
    spec_r = pl.BlockSpec((1, L * P, LQP), lambda n, j: (n, 0, 0))
    spec_o = pl.BlockSpec((1, LQP, 128), lambda n, j: (n, 0, j))
    oshape = jax.ShapeDtypeStruct((N, LQP, ROWS), jnp.int32)
    wshape = jax.ShapeDtypeStruct((N, LQP, ROWS), jnp.float32)
    idx_o, wgt_o = pl.pallas_call(
        _prep_body,
        grid=(N, H // 2),
        in_specs=[spec_q, spec_w, spec_w, spec_w, spec_b, spec_b, spec_b,
                  spec_r, spec_r, spec_r, spec_r],
        out_specs=[spec_o, spec_o],
        out_shape=[oshape, wshape],
    )(qT, WaT, WoxT, WoyT, baT, boxT, boyT, cxb, cyb, wwb, hhb)

    idx_t = idx_o.reshape(N * LQP, 4, 128)
    wgt_t = wgt_o.reshape(N * LQP, ROWS)

    vtab = _matmul(value.reshape(N * LV, C), W_val, b_val, bm=1024,
                   out_dtype=jnp.bfloat16).reshape(N * LV * H, CH)

    sampled = _sc_gather(vtab, idx_t, wgt_t)

    cperm = np.concatenate(
        [np.concatenate([h * CH + 2 * np.arange(16),
                         h * CH + 2 * np.arange(16) + 1]) for h in range(H)])
    out = _matmul(sampled, W_out[cperm, :], b_out,
                  bm=LQP).reshape(N, LQP, C)[:, :LQ]
    return out


# speedup vs baseline: 1.1253x; 1.1253x over previous
import functools

import numpy as np
import jax
import jax.numpy as jnp
from jax import lax
from jax.experimental import pallas as pl
from jax.experimental.pallas import tpu as pltpu
from jax.experimental.pallas import tpu_sc as plsc

H = 8
L = 4
P = 4
LV = 21760
N = 4
LQ = 900
LQP = 960
C = 256
CH = 32
NW = 32
PAIRS_PER_W = (N * LQP) // NW
CHUNK = 8
NCHUNK = PAIRS_PER_W // CHUNK
GRAN = 1
GPC = CHUNK // GRAN
ROWS = H * L * P * 4


def _prep_body(qT, Wa, Wox, Woy, ba, box, boy, cx, cy, ww, hb, iref, wref):
    n = pl.program_id(0)
    h = pl.program_id(1)
    q = qT[0]
    att = jnp.dot(Wa[0], q, preferred_element_type=jnp.float32) + ba[0][:, 0:1]
    att = att - jnp.max(att, axis=0, keepdims=True)
    att = jnp.exp(att)
    att = att / jnp.sum(att, axis=0, keepdims=True)
    ox = jnp.dot(Wox[0], q, preferred_element_type=jnp.float32) + box[0][:, 0:1]
    oy = jnp.dot(Woy[0], q, preferred_element_type=jnp.float32) + boy[0][:, 0:1]
    locx = cx[0] + ox * 0.125 * ww[0]
    locy = cy[0] + oy * 0.125 * hb[0]
    lrow = lax.broadcasted_iota(jnp.int32, (L * P, LQP), 0) // P
    Wl = jnp.where(lrow == 0, 128, jnp.where(lrow == 1, 64,
                                             jnp.where(lrow == 2, 32, 16)))
    offl = jnp.where(lrow == 0, 0, jnp.where(lrow == 1, 16384,
                                             jnp.where(lrow == 2, 20480, 21504)))
    Wlf = Wl.astype(jnp.float32)
    x = locx * Wlf - 0.5
    y = locy * Wlf - 0.5
    x0 = jnp.floor(x)
    y0 = jnp.floor(y)
    fx = x - x0
    fy = y - y0
    base = n * (LV * H) + h
    ci = 0
    for dy in (0, 1):
        wy = fy if dy else 1.0 - fy
        for dx in (0, 1):
            wx = fx if dx else 1.0 - fx
            xi = x0 + dx
            yi = y0 + dy
            valid = ((xi >= 0) & (xi <= Wlf - 1.0)
                     & (yi >= 0) & (yi <= Wlf - 1.0))
            xc = jnp.clip(xi, 0.0, Wlf - 1.0).astype(jnp.int32)
            yc = jnp.clip(yi, 0.0, Wlf - 1.0).astype(jnp.int32)
            rowid = (offl + yc * Wl + xc) * H + base
            wcorn = att * (wx * wy) * valid.astype(jnp.float32)
            iref[0, 0, ci] = rowid
            wref[0, 0, ci] = wcorn
            ci += 1


def _mm_body(x, w, b, o):
    o[...] = (jnp.dot(x[...], w[...], preferred_element_type=jnp.float32)
              + b[...]).astype(o.dtype)


def _matmul(x, w, b, bm, out_dtype=jnp.float32):
    m = x.shape[0]
    assert m % bm == 0
    return pl.pallas_call(
        _mm_body,
        grid=(m // bm,),
        in_specs=[
            pl.BlockSpec((bm, x.shape[1]), lambda i: (i, 0)),
            pl.BlockSpec((w.shape[0], w.shape[1]), lambda i: (0, 0)),
            pl.BlockSpec((1, w.shape[1]), lambda i: (0, 0)),
        ],
        out_specs=pl.BlockSpec((bm, w.shape[1]), lambda i: (i, 0)),
        out_shape=jax.ShapeDtypeStruct((m, w.shape[1]), out_dtype),
    )(x, w, b.reshape(1, -1))


def _sc_body(vtab, idxh, wgth, outh, ibuf, wbuf, rows, ob,
             i0, i1, g0, g1, o0, o1):
    isems = (i0, i1)
    gsems = (g0, g1)
    osems = (o0, o1)
    wid = lax.axis_index("s") * 2 + lax.axis_index("c")
    base = wid * PAIRS_PER_W
    zero = jnp.zeros((16,), jnp.float32)

    def load_chunk(c, cb):
        pltpu.async_copy(idxh.at[pl.ds(base + c * CHUNK, CHUNK)],
                         ibuf.at[cb], isems[cb])
        pltpu.async_copy(wgth.at[pl.ds(base + c * CHUNK, CHUNK)],
                         wbuf.at[cb], isems[cb])

    def drain_chunk(cb):
        pltpu.make_async_copy(idxh.at[pl.ds(0, CHUNK)],
                              ibuf.at[cb], isems[cb]).wait()
        pltpu.make_async_copy(wgth.at[pl.ds(0, CHUNK)],
                              wbuf.at[cb], isems[cb]).wait()

    def fire(g, cb, rb):
        for u in range(GRAN):
            tin = g * GRAN + u
            for jj in range(4):
                pltpu.async_copy(
                    vtab.at[ibuf.at[cb, tin, jj]],
                    rows.at[rb].at[pl.ds((u * 4 + jj) * 128, 128)],
                    gsems[rb])

    def drain_gather(rb):
        pltpu.make_async_copy(vtab.at[pl.ds(0, GRAN * ROWS)],
                              rows.at[rb], gsems[rb]).wait()

    def drain_out(b):
        pltpu.make_async_copy(outh.at[pl.ds(0, GRAN)], ob.at[b],
                              osems[b]).wait()

    def compute(g, c, cb, rb):
        for u in range(GRAN):
            tin = g * GRAN + u
            for i in range(C // 16):
                ob[rb, u, pl.ds(i * 16, 16)] = zero

            def grp_body(gg, carry, u=u, tin=tin):
                wvec = wbuf[cb, tin, pl.ds(gg * 16, 16)]
                a0 = zero
                a1 = zero
                for k in range(16):
                    r = u * ROWS + gg * 16 + k
                    e, o = plsc.unpack(rows[rb, r, 0:32],
                                       format=plsc.PackFormat.INTERLEAVED,
                                       preferred_element_type=jnp.float32)
                    a0 = a0 + wvec[k] * e
                    a1 = a1 + wvec[k] * o
                h = gg // 4
                ob[rb, u, pl.ds(h * CH, 16)] = (ob[rb, u, pl.ds(h * CH, 16)]
                                                + a0)
                ob[rb, u, pl.ds(h * CH + 16, 16)] = (
                    ob[rb, u, pl.ds(h * CH + 16, 16)] + a1)
                return carry

            lax.fori_loop(0, ROWS // 16, grp_body, 0)
            pltpu.async_copy(ob.at[rb, u],
                             outh.at[base + c * CHUNK + tin], osems[rb])

    def chunk_pairs(c, cb, last):
        for g in range(GPC):
            rb = g % 2
            if g < GPC - 1:
                fire(g + 1, cb, (g + 1) % 2)
            drain_gather(rb)
            drain_out(rb)
            compute(g, c, cb, rb)
        if not last:
            drain_chunk(cb ^ 1)
            fire(0, cb ^ 1, 0)

    for b in range(2):
        pltpu.async_copy(outh.at[pl.ds(0, GRAN)], ob.at[b], osems[b])
    load_chunk(0, 0)
    drain_chunk(0)
    fire(0, 0, 0)

    def dchunk_body(g, carry):
        load_chunk(2 * g + 1, 1)
        chunk_pairs(2 * g, 0, False)
        load_chunk(2 * g + 2, 0)
        chunk_pairs(2 * g + 1, 1, False)
        return carry

    lax.fori_loop(0, (NCHUNK - 1) // 2, dchunk_body, 0)
    chunk_pairs(NCHUNK - 1, 0, True)
    drain_out(0)
    drain_out(1)


def _sc_gather(vtab, idx_t, wgt_t):
    k = functools.partial(
        pl.kernel,
        out_type=jax.ShapeDtypeStruct((N * LQP, C), jnp.float32),
        mesh=plsc.VectorSubcoreMesh(core_axis_name="c", subcore_axis_name="s"),
        scratch_types=[
            pltpu.VMEM((2, CHUNK, 4, 128), jnp.int32),
            pltpu.VMEM((2, CHUNK, ROWS), jnp.float32),
            pltpu.VMEM((2, GRAN * ROWS, CH), jnp.bfloat16),
            pltpu.VMEM((2, GRAN, C), jnp.float32),
            pltpu.SemaphoreType.DMA,
            pltpu.SemaphoreType.DMA,
            pltpu.SemaphoreType.DMA,
            pltpu.SemaphoreType.DMA,
            pltpu.SemaphoreType.DMA,
            pltpu.SemaphoreType.DMA,
        ],
        compiler_params=pltpu.CompilerParams(use_tc_tiling_on_sc=False,
                                            needs_layout_passes=False),
    )(_sc_body)
    return k(vtab, idx_t, wgt_t)


def kernel(query, ref_points, value, value_spatial_shapes,
           W_off, b_off, W_attn, b_attn, W_val, b_val, W_out, b_out):
    qT = jnp.pad(query, ((0, 0), (0, LQP - LQ), (0, 0))).transpose(0, 2, 1)
    WaT = W_attn.T.reshape(H, L * P, C)
    WoT = W_off.T.reshape(H, L * P, 2, C)
    WoxT = WoT[:, :, 0, :]
    WoyT = WoT[:, :, 1, :]
    baT = jnp.broadcast_to(b_attn.reshape(H, L * P, 1), (H, L * P, 128))
    bo = b_off.reshape(H, L * P, 2)
    boxT = jnp.broadcast_to(bo[:, :, 0:1], (H, L * P, 128))
    boyT = jnp.broadcast_to(bo[:, :, 1:2], (H, L * P, 128))
    rp = jnp.pad(ref_points,
                 ((0, 0), (0, LQP - LQ), (0, 0), (0, 0))).transpose(0, 2, 3, 1)
    cxb = jnp.repeat(rp[:, :, 0, :], P, axis=1)
    cyb = jnp.repeat(rp[:, :, 1, :], P, axis=1)
    wwb = jnp.repeat(rp[:, :, 2, :], P, axis=1)
    hhb = jnp.repeat(rp[:, :, 3, :], P, axis=1)

    spec_q = pl.BlockSpec((1, C, LQP), lambda n, h: (n, 0, 0))
    spec_w = pl.BlockSpec((1, L * P, C), lambda n, h: (h, 0, 0))
    spec_b = pl.BlockSpec((1, L * P, 128), lambda n, h: (h, 0, 0))
    spec_r = pl.BlockSpec((1, L * P, LQP), lambda n, h: (n, 0, 0))
    spec_o = pl.BlockSpec((1, 1, 4, L * P, LQP), lambda n, h: (n, h, 0, 0, 0))
    oshape = jax.ShapeDtypeStruct((N, H, 4, L * P, LQP), jnp.int32)
    wshape = jax.ShapeDtypeStruct((N, H, 4, L * P, LQP), jnp.float32)
    idx_o, wgt_o = pl.pallas_call(
        _prep_body,
        grid=(N, H),
        in_specs=[spec_q, spec_w, spec_w, spec_w, spec_b, spec_b, spec_b,
                  spec_r, spec_r, spec_r, spec_r],
        out_specs=[spec_o, spec_o],
        out_shape=[oshape, wshape],
    )(qT, WaT, WoxT, WoyT, baT, boxT, boyT, cxb, cyb, wwb, hhb)

    idx_t = idx_o.transpose(0, 4, 1, 2, 3).reshape(N * LQP, 4, 128)
    wgt_t = wgt_o.transpose(0, 4, 1, 2, 3).reshape(N * LQP, ROWS)

    vtab = _matmul(value.reshape(N * LV, C), W_val, b_val, bm=1024,
                   out_dtype=jnp.bfloat16).reshape(N * LV * H, CH)

    sampled = _sc_gather(vtab, idx_t, wgt_t)

    cperm = np.concatenate(
        [np.concatenate([h * CH + 2 * np.arange(16),
                         h * CH + 2 * np.arange(16) + 1]) for h in range(H)])
    out = _matmul(sampled, W_out[cperm, :], b_out,
                  bm=LQP).reshape(N, LQP, C)[:, :LQ]
    return out


# speedup vs baseline: 1.1551x; 1.0266x over previous
import functools

import numpy as np
import jax
import jax.numpy as jnp
from jax import lax
from jax.experimental import pallas as pl
from jax.experimental.pallas import tpu as pltpu
from jax.experimental.pallas import tpu_sc as plsc

H = 8
L = 4
P = 4
LV = 21760
N = 4
LQ = 900
LQP = 960
C = 256
CH = 32
NW = 32
PAIRS_PER_W = (N * LQP) // NW
CHUNK = 8
NCHUNK = PAIRS_PER_W // CHUNK
NCH0 = 14
NCH1 = 16
GRAN = 1
GPC = CHUNK // GRAN
ROWS = H * L * P * 4


def _prep_body(qT, Wa, Wox, Woy, ba, box, boy, cx, cy, ww, hb, iref, wref):
    n = pl.program_id(0)
    h = pl.program_id(1)
    q = qT[0]
    att = jnp.dot(Wa[0], q, preferred_element_type=jnp.float32) + ba[0][:, 0:1]
    att = att - jnp.max(att, axis=0, keepdims=True)
    att = jnp.exp(att)
    att = att / jnp.sum(att, axis=0, keepdims=True)
    ox = jnp.dot(Wox[0], q, preferred_element_type=jnp.float32) + box[0][:, 0:1]
    oy = jnp.dot(Woy[0], q, preferred_element_type=jnp.float32) + boy[0][:, 0:1]
    locx = cx[0] + ox * 0.125 * ww[0]
    locy = cy[0] + oy * 0.125 * hb[0]
    lrow = lax.broadcasted_iota(jnp.int32, (L * P, LQP), 0) // P
    Wl = jnp.where(lrow == 0, 128, jnp.where(lrow == 1, 64,
                                             jnp.where(lrow == 2, 32, 16)))
    offl = jnp.where(lrow == 0, 0, jnp.where(lrow == 1, 16384,
                                             jnp.where(lrow == 2, 20480, 21504)))
    Wlf = Wl.astype(jnp.float32)
    x = locx * Wlf - 0.5
    y = locy * Wlf - 0.5
    x0 = jnp.floor(x)
    y0 = jnp.floor(y)
    fx = x - x0
    fy = y - y0
    base = n * (LV * H) + h
    ci = 0
    for dy in (0, 1):
        wy = fy if dy else 1.0 - fy
        for dx in (0, 1):
            wx = fx if dx else 1.0 - fx
            xi = x0 + dx
            yi = y0 + dy
            valid = ((xi >= 0) & (xi <= Wlf - 1.0)
                     & (yi >= 0) & (yi <= Wlf - 1.0))
            xc = jnp.clip(xi, 0.0, Wlf - 1.0).astype(jnp.int32)
            yc = jnp.clip(yi, 0.0, Wlf - 1.0).astype(jnp.int32)
            rowid = (offl + yc * Wl + xc) * H + base
            wcorn = att * (wx * wy) * valid.astype(jnp.float32)
            iref[0, 0, ci] = rowid
            wref[0, 0, ci] = wcorn
            ci += 1


def _mm_body(x, w, b, o):
    o[...] = (jnp.dot(x[...], w[...], preferred_element_type=jnp.float32)
              + b[...]).astype(o.dtype)


def _matmul(x, w, b, bm, out_dtype=jnp.float32):
    m = x.shape[0]
    assert m % bm == 0
    return pl.pallas_call(
        _mm_body,
        grid=(m // bm,),
        in_specs=[
            pl.BlockSpec((bm, x.shape[1]), lambda i: (i, 0)),
            pl.BlockSpec((w.shape[0], w.shape[1]), lambda i: (0, 0)),
            pl.BlockSpec((1, w.shape[1]), lambda i: (0, 0)),
        ],
        out_specs=pl.BlockSpec((bm, w.shape[1]), lambda i: (i, 0)),
        out_shape=jax.ShapeDtypeStruct((m, w.shape[1]), out_dtype),
    )(x, w, b.reshape(1, -1))


def _sc_body(vtab, idxh, wgth, outh, ibuf, wbuf, rows, ob,
             i0, i1, g0, g1, o0, o1):
    isems = (i0, i1)
    gsems = (g0, g1)
    osems = (o0, o1)
    c_ax = lax.axis_index("c")
    base = lax.axis_index("s") * (2 * PAIRS_PER_W) + c_ax * (NCH0 * CHUNK)
    nchunk = jnp.where(c_ax == 0, NCH0, NCH1)
    zero = jnp.zeros((16,), jnp.float32)

    def load_chunk(c, cb):
        pltpu.async_copy(idxh.at[pl.ds(base + c * CHUNK, CHUNK)],
                         ibuf.at[cb], isems[cb])
        pltpu.async_copy(wgth.at[pl.ds(base + c * CHUNK, CHUNK)],
                         wbuf.at[cb], isems[cb])

    def drain_chunk(cb):
        pltpu.make_async_copy(idxh.at[pl.ds(0, CHUNK)],
                              ibuf.at[cb], isems[cb]).wait()
        pltpu.make_async_copy(wgth.at[pl.ds(0, CHUNK)],
                              wbuf.at[cb], isems[cb]).wait()

    def fire(g, cb, rb):
        for u in range(GRAN):
            tin = g * GRAN + u
            for jj in range(4):
                pltpu.async_copy(
                    vtab.at[ibuf.at[cb, tin, jj]],
                    rows.at[rb].at[pl.ds((u * 4 + jj) * 128, 128)],
                    gsems[rb])

    def drain_gather(rb):
        pltpu.make_async_copy(vtab.at[pl.ds(0, GRAN * ROWS)],
                              rows.at[rb], gsems[rb]).wait()

    def drain_out(b):
        pltpu.make_async_copy(outh.at[pl.ds(0, GRAN)], ob.at[b],
                              osems[b]).wait()

    def compute(g, c, cb, rb):
        for u in range(GRAN):
            tin = g * GRAN + u
            for i in range(C // 16):
                ob[rb, u, pl.ds(i * 16, 16)] = zero

            def grp_body(gg, carry, u=u, tin=tin):
                wvec = wbuf[cb, tin, pl.ds(gg * 16, 16)]
                a0 = zero
                a1 = zero
                for k in range(16):
                    r = u * ROWS + gg * 16 + k
                    e, o = plsc.unpack(rows[rb, r, 0:32],
                                       format=plsc.PackFormat.INTERLEAVED,
                                       preferred_element_type=jnp.float32)
                    a0 = a0 + wvec[k] * e
                    a1 = a1 + wvec[k] * o
                h = gg // 4
                ob[rb, u, pl.ds(h * CH, 16)] = (ob[rb, u, pl.ds(h * CH, 16)]
                                                + a0)
                ob[rb, u, pl.ds(h * CH + 16, 16)] = (
                    ob[rb, u, pl.ds(h * CH + 16, 16)] + a1)
                return carry

            lax.fori_loop(0, ROWS // 16, grp_body, 0)
            pltpu.async_copy(ob.at[rb, u],
                             outh.at[base + c * CHUNK + tin], osems[rb])

    def chunk_pairs(c, cb, last):
        for g in range(GPC):
            rb = g % 2
            if g < GPC - 1:
                fire(g + 1, cb, (g + 1) % 2)
            drain_gather(rb)
            drain_out(rb)
            compute(g, c, cb, rb)
        if not last:
            drain_chunk(cb ^ 1)
            fire(0, cb ^ 1, 0)

    for b in range(2):
        pltpu.async_copy(outh.at[pl.ds(0, GRAN)], ob.at[b], osems[b])
    load_chunk(0, 0)
    drain_chunk(0)
    fire(0, 0, 0)

    def dchunk_body(g, carry):
        load_chunk(2 * g + 1, 1)
        chunk_pairs(2 * g, 0, False)
        load_chunk(2 * g + 2, 0)
        chunk_pairs(2 * g + 1, 1, False)
        return carry

    lax.fori_loop(0, (nchunk - 2) // 2, dchunk_body, 0)
    load_chunk(nchunk - 1, 1)
    chunk_pairs(nchunk - 2, 0, False)
    chunk_pairs(nchunk - 1, 1, True)
    drain_out(0)
    drain_out(1)


def _sc_gather(vtab, idx_t, wgt_t):
    k = functools.partial(
        pl.kernel,
        out_type=jax.ShapeDtypeStruct((N * LQP, C), jnp.float32),
        mesh=plsc.VectorSubcoreMesh(core_axis_name="c", subcore_axis_name="s"),
        scratch_types=[
            pltpu.VMEM((2, CHUNK, 4, 128), jnp.int32),
            pltpu.VMEM((2, CHUNK, ROWS), jnp.float32),
            pltpu.VMEM((2, GRAN * ROWS, CH), jnp.bfloat16),
            pltpu.VMEM((2, GRAN, C), jnp.float32),
            pltpu.SemaphoreType.DMA,
            pltpu.SemaphoreType.DMA,
            pltpu.SemaphoreType.DMA,
            pltpu.SemaphoreType.DMA,
            pltpu.SemaphoreType.DMA,
            pltpu.SemaphoreType.DMA,
        ],
        compiler_params=pltpu.CompilerParams(use_tc_tiling_on_sc=False,
                                            needs_layout_passes=False),
    )(_sc_body)
    return k(vtab, idx_t, wgt_t)


def kernel(query, ref_points, value, value_spatial_shapes,
           W_off, b_off, W_attn, b_attn, W_val, b_val, W_out, b_out):
    qT = jnp.pad(query, ((0, 0), (0, LQP - LQ), (0, 0))).transpose(0, 2, 1)
    WaT = W_attn.T.reshape(H, L * P, C)
    WoT = W_off.T.reshape(H, L * P, 2, C)
    WoxT = WoT[:, :, 0, :]
    WoyT = WoT[:, :, 1, :]
    baT = jnp.broadcast_to(b_attn.reshape(H, L * P, 1), (H, L * P, 128))
    bo = b_off.reshape(H, L * P, 2)
    boxT = jnp.broadcast_to(bo[:, :, 0:1], (H, L * P, 128))
    boyT = jnp.broadcast_to(bo[:, :, 1:2], (H, L * P, 128))
    rp = jnp.pad(ref_points,
                 ((0, 0), (0, LQP - LQ), (0, 0), (0, 0))).transpose(0, 2, 3, 1)
    cxb = jnp.repeat(rp[:, :, 0, :], P, axis=1)
    cyb = jnp.repeat(rp[:, :, 1, :], P, axis=1)
    wwb = jnp.repeat(rp[:, :, 2, :], P, axis=1)
    hhb = jnp.repeat(rp[:, :, 3, :], P, axis=1)

    spec_q = pl.BlockSpec((1, C, LQP), lambda n, h: (n, 0, 0))
    spec_w = pl.BlockSpec((1, L * P, C), lambda n, h: (h, 0, 0))
    spec_b = pl.BlockSpec((1, L * P, 128), lambda n, h: (h, 0, 0))
    spec_r = pl.BlockSpec((1, L * P, LQP), lambda n, h: (n, 0, 0))
    spec_o = pl.BlockSpec((1, 1, 4, L * P, LQP), lambda n, h: (n, h, 0, 0, 0))
    oshape = jax.ShapeDtypeStruct((N, H, 4, L * P, LQP), jnp.int32)
    wshape = jax.ShapeDtypeStruct((N, H, 4, L * P, LQP), jnp.float32)
    idx_o, wgt_o = pl.pallas_call(
        _prep_body,
        grid=(N, H),
        in_specs=[spec_q, spec_w, spec_w, spec_w, spec_b, spec_b, spec_b,
                  spec_r, spec_r, spec_r, spec_r],
        out_specs=[spec_o, spec_o],
        out_shape=[oshape, wshape],
    )(qT, WaT, WoxT, WoyT, baT, boxT, boyT, cxb, cyb, wwb, hhb)

    idx_t = idx_o.transpose(0, 4, 1, 3, 2).reshape(N * LQP, 4, 128)
    wgt_t = wgt_o.transpose(0, 4, 1, 3, 2).reshape(N * LQP, ROWS)

    vtab = _matmul(value.reshape(N * LV, C), W_val, b_val, bm=1024,
                   out_dtype=jnp.bfloat16).reshape(N * LV * H, CH)

    sampled = _sc_gather(vtab, idx_t, wgt_t)

    cperm = np.concatenate(
        [np.concatenate([h * CH + 2 * np.arange(16),
                         h * CH + 2 * np.arange(16) + 1]) for h in range(H)])
    out = _matmul(sampled, W_out[cperm, :], b_out,
                  bm=LQP).reshape(N, LQP, C)[:, :LQ]
    return out


# speedup vs baseline: 1.2047x; 1.0429x over previous
import functools

import numpy as np
import jax
import jax.numpy as jnp
from jax import lax
from jax.experimental import pallas as pl
from jax.experimental.pallas import tpu as pltpu
from jax.experimental.pallas import tpu_sc as plsc

H = 8
L = 4
P = 4
LV = 21760
N = 4
LQ = 900
LQP = 960
C = 256
CH = 32
NW = 32
PAIRS_PER_W = (N * LQP) // NW
CHUNK = 8
NCHUNK = PAIRS_PER_W // CHUNK
NCH0 = 16
NCH1 = 14
GRAN = 1
GPC = CHUNK // GRAN
ROWS = H * L * P * 4


def _prep_body(qT, Wa, Wox, Woy, ba, box, boy, cx, cy, ww, hb, iref, wref):
    n = pl.program_id(0)
    h = pl.program_id(1)
    q = qT[0]
    att = jnp.dot(Wa[0], q, preferred_element_type=jnp.float32) + ba[0][:, 0:1]
    att = att - jnp.max(att, axis=0, keepdims=True)
    att = jnp.exp(att)
    att = att / jnp.sum(att, axis=0, keepdims=True)
    ox = jnp.dot(Wox[0], q, preferred_element_type=jnp.float32) + box[0][:, 0:1]
    oy = jnp.dot(Woy[0], q, preferred_element_type=jnp.float32) + boy[0][:, 0:1]
    locx = cx[0] + ox * 0.125 * ww[0]
    locy = cy[0] + oy * 0.125 * hb[0]
    lrow = lax.broadcasted_iota(jnp.int32, (L * P, LQP), 0) // P
    Wl = jnp.where(lrow == 0, 128, jnp.where(lrow == 1, 64,
                                             jnp.where(lrow == 2, 32, 16)))
    offl = jnp.where(lrow == 0, 0, jnp.where(lrow == 1, 16384,
                                             jnp.where(lrow == 2, 20480, 21504)))
    Wlf = Wl.astype(jnp.float32)
    x = locx * Wlf - 0.5
    y = locy * Wlf - 0.5
    x0 = jnp.floor(x)
    y0 = jnp.floor(y)
    fx = x - x0
    fy = y - y0
    base = n * (LV * H) + h
    ci = 0
    for dy in (0, 1):
        wy = fy if dy else 1.0 - fy
        for dx in (0, 1):
            wx = fx if dx else 1.0 - fx
            xi = x0 + dx
            yi = y0 + dy
            valid = ((xi >= 0) & (xi <= Wlf - 1.0)
                     & (yi >= 0) & (yi <= Wlf - 1.0))
            xc = jnp.clip(xi, 0.0, Wlf - 1.0).astype(jnp.int32)
            yc = jnp.clip(yi, 0.0, Wlf - 1.0).astype(jnp.int32)
            rowid = (offl + yc * Wl + xc) * H + base
            wcorn = att * (wx * wy) * valid.astype(jnp.float32)
            iref[0, 0, ci] = rowid
            wref[0, 0, ci] = wcorn
            ci += 1


def _mm_body(x, w, b, o):
    o[...] = (jnp.dot(x[...], w[...], preferred_element_type=jnp.float32)
              + b[...]).astype(o.dtype)


def _matmul(x, w, b, bm, out_dtype=jnp.float32):
    m = x.shape[0]
    assert m % bm == 0
    return pl.pallas_call(
        _mm_body,
        grid=(m // bm,),
        in_specs=[
            pl.BlockSpec((bm, x.shape[1]), lambda i: (i, 0)),
            pl.BlockSpec((w.shape[0], w.shape[1]), lambda i: (0, 0)),
            pl.BlockSpec((1, w.shape[1]), lambda i: (0, 0)),
        ],
        out_specs=pl.BlockSpec((bm, w.shape[1]), lambda i: (i, 0)),
        out_shape=jax.ShapeDtypeStruct((m, w.shape[1]), out_dtype),
    )(x, w, b.reshape(1, -1))


def _sc_body(vtab, idxh, wgth, outh, ibuf, wbuf, rows, ob,
             i0, i1, g0, g1, o0, o1):
    isems = (i0, i1)
    gsems = (g0, g1)
    osems = (o0, o1)
    c_ax = lax.axis_index("c")
    base = lax.axis_index("s") * (2 * PAIRS_PER_W) + c_ax * (NCH0 * CHUNK)
    nchunk = jnp.where(c_ax == 0, NCH0, NCH1)
    zero = jnp.zeros((16,), jnp.float32)

    def load_chunk(c, cb):
        pltpu.async_copy(idxh.at[pl.ds(base + c * CHUNK, CHUNK)],
                         ibuf.at[cb], isems[cb])
        pltpu.async_copy(wgth.at[pl.ds(base + c * CHUNK, CHUNK)],
                         wbuf.at[cb], isems[cb])

    def drain_chunk(cb):
        pltpu.make_async_copy(idxh.at[pl.ds(0, CHUNK)],
                              ibuf.at[cb], isems[cb]).wait()
        pltpu.make_async_copy(wgth.at[pl.ds(0, CHUNK)],
                              wbuf.at[cb], isems[cb]).wait()

    def fire(g, cb, rb):
        for u in range(GRAN):
            tin = g * GRAN + u
            for jj in range(4):
                pltpu.async_copy(
                    vtab.at[ibuf.at[cb, tin, jj]],
                    rows.at[rb].at[pl.ds((u * 4 + jj) * 128, 128)],
                    gsems[rb])

    def drain_gather(rb):
        pltpu.make_async_copy(vtab.at[pl.ds(0, GRAN * ROWS)],
                              rows.at[rb], gsems[rb]).wait()

    def drain_out(b):
        pltpu.make_async_copy(outh.at[pl.ds(0, GRAN)], ob.at[b],
                              osems[b]).wait()

    def compute(g, c, cb, rb):
        for u in range(GRAN):
            tin = g * GRAN + u
            for i in range(C // 16):
                ob[rb, u, pl.ds(i * 16, 16)] = zero

            def grp_body(gg, carry, u=u, tin=tin):
                wvec = wbuf[cb, tin, pl.ds(gg * 16, 16)]
                a0 = zero
                a1 = zero
                for k in range(16):
                    r = u * ROWS + gg * 16 + k
                    e, o = plsc.unpack(rows[rb, r, 0:32],
                                       format=plsc.PackFormat.INTERLEAVED,
                                       preferred_element_type=jnp.float32)
                    a0 = a0 + wvec[k] * e
                    a1 = a1 + wvec[k] * o
                h = gg // 4
                ob[rb, u, pl.ds(h * CH, 16)] = (ob[rb, u, pl.ds(h * CH, 16)]
                                                + a0)
                ob[rb, u, pl.ds(h * CH + 16, 16)] = (
                    ob[rb, u, pl.ds(h * CH + 16, 16)] + a1)
                return carry

            lax.fori_loop(0, ROWS // 16, grp_body, 0)
            pltpu.async_copy(ob.at[rb, u],
                             outh.at[base + c * CHUNK + tin], osems[rb])

    def chunk_pairs(c, cb, last):
        for g in range(GPC):
            rb = g % 2
            if g < GPC - 1:
                fire(g + 1, cb, (g + 1) % 2)
            drain_gather(rb)
            drain_out(rb)
            compute(g, c, cb, rb)
        if not last:
            drain_chunk(cb ^ 1)
            fire(0, cb ^ 1, 0)

    for b in range(2):
        pltpu.async_copy(outh.at[pl.ds(0, GRAN)], ob.at[b], osems[b])
    load_chunk(0, 0)
    drain_chunk(0)
    fire(0, 0, 0)

    def dchunk_body(g, carry):
        load_chunk(2 * g + 1, 1)
        chunk_pairs(2 * g, 0, False)
        load_chunk(2 * g + 2, 0)
        chunk_pairs(2 * g + 1, 1, False)
        return carry

    lax.fori_loop(0, (nchunk - 2) // 2, dchunk_body, 0)
    load_chunk(nchunk - 1, 1)
    chunk_pairs(nchunk - 2, 0, False)
    chunk_pairs(nchunk - 1, 1, True)
    drain_out(0)
    drain_out(1)


def _sc_gather(vtab, idx_t, wgt_t):
    k = functools.partial(
        pl.kernel,
        out_type=jax.ShapeDtypeStruct((N * LQP, C), jnp.float32),
        mesh=plsc.VectorSubcoreMesh(core_axis_name="c", subcore_axis_name="s"),
        scratch_types=[
            pltpu.VMEM((2, CHUNK, 4, 128), jnp.int32),
            pltpu.VMEM((2, CHUNK, ROWS), jnp.float32),
            pltpu.VMEM((2, GRAN * ROWS, CH), jnp.bfloat16),
            pltpu.VMEM((2, GRAN, C), jnp.float32),
            pltpu.SemaphoreType.DMA,
            pltpu.SemaphoreType.DMA,
            pltpu.SemaphoreType.DMA,
            pltpu.SemaphoreType.DMA,
            pltpu.SemaphoreType.DMA,
            pltpu.SemaphoreType.DMA,
        ],
        compiler_params=pltpu.CompilerParams(use_tc_tiling_on_sc=False,
                                            needs_layout_passes=False),
    )(_sc_body)
    return k(vtab, idx_t, wgt_t)


def kernel(query, ref_points, value, value_spatial_shapes,
           W_off, b_off, W_attn, b_attn, W_val, b_val, W_out, b_out):
    qT = jnp.pad(query, ((0, 0), (0, LQP - LQ), (0, 0))).transpose(0, 2, 1)
    WaT = W_attn.T.reshape(H, L * P, C)
    WoT = W_off.T.reshape(H, L * P, 2, C)
    WoxT = WoT[:, :, 0, :]
    WoyT = WoT[:, :, 1, :]
    baT = jnp.broadcast_to(b_attn.reshape(H, L * P, 1), (H, L * P, 128))
    bo = b_off.reshape(H, L * P, 2)
    boxT = jnp.broadcast_to(bo[:, :, 0:1], (H, L * P, 128))
    boyT = jnp.broadcast_to(bo[:, :, 1:2], (H, L * P, 128))
    rp = jnp.pad(ref_points,
                 ((0, 0), (0, LQP - LQ), (0, 0), (0, 0))).transpose(0, 2, 3, 1)
    cxb = jnp.repeat(rp[:, :, 0, :], P, axis=1)
    cyb = jnp.repeat(rp[:, :, 1, :], P, axis=1)
    wwb = jnp.repeat(rp[:, :, 2, :], P, axis=1)
    hhb = jnp.repeat(rp[:, :, 3, :], P, axis=1)

    spec_q = pl.BlockSpec((1, C, LQP), lambda n, h: (n, 0, 0))
    spec_w = pl.BlockSpec((1, L * P, C), lambda n, h: (h, 0, 0))
    spec_b = pl.BlockSpec((1, L * P, 128), lambda n, h: (h, 0, 0))
    spec_r = pl.BlockSpec((1, L * P, LQP), lambda n, h: (n, 0, 0))
    spec_o = pl.BlockSpec((1, 1, 4, L * P, LQP), lambda n, h: (n, h, 0, 0, 0))
    oshape = jax.ShapeDtypeStruct((N, H, 4, L * P, LQP), jnp.int32)
    wshape = jax.ShapeDtypeStruct((N, H, 4, L * P, LQP), jnp.float32)
    idx_o, wgt_o = pl.pallas_call(
        _prep_body,
        grid=(N, H),
        in_specs=[spec_q, spec_w, spec_w, spec_w, spec_b, spec_b, spec_b,
                  spec_r, spec_r, spec_r, spec_r],
        out_specs=[spec_o, spec_o],
        out_shape=[oshape, wshape],
    )(qT, WaT, WoxT, WoyT, baT, boxT, boyT, cxb, cyb, wwb, hhb)

    idx_t = idx_o.transpose(0, 4, 1, 3, 2).reshape(N * LQP, 4, 128)
    wgt_t = wgt_o.transpose(0, 4, 1, 3, 2).reshape(N * LQP, ROWS)

    vtab = _matmul(value.reshape(N * LV, C), W_val, b_val, bm=1024,
                   out_dtype=jnp.bfloat16).reshape(N * LV * H, CH)

    sampled = _sc_gather(vtab, idx_t, wgt_t)

    cperm = np.concatenate(
        [np.concatenate([h * CH + 2 * np.arange(16),
                         h * CH + 2 * np.arange(16) + 1]) for h in range(H)])
    out = _matmul(sampled, W_out[cperm, :], b_out,
                  bm=LQP).reshape(N, LQP, C)[:, :LQ]
    return out
